# Initial kernel scaffold; baseline (speedup 1.0000x reference)
import jax
import jax.numpy as jnp
from jax.experimental import pallas as pl


def kernel(inputs, tables, ctr_W1, ctr_b1, ctr_W2, ctr_b2, cvr_W1, cvr_b1, cvr_W2, cvr_b2):
    raise NotImplementedError("write your pallas kernel here")


# speedup vs baseline: 2.2044x; 2.2044x over previous
import functools

import jax
import jax.numpy as jnp
from jax import lax
from jax.experimental import pallas as pl
from jax.experimental.pallas import tpu as pltpu
from jax.experimental.pallas import tpu_sc as plsc

B, F, V, D = 4096, 26, 100000, 32
INPUT_DIM = F * D
H1 = 128

_SC_INFO = plsc.get_sparse_core_info()
_NC = _SC_INFO.num_cores
_NS = _SC_INFO.num_subcores
_NW = _NC * _NS
_N_PER_W = (B * F) // _NW
_LANES = 16


def _gather_body(idx_hbm, table_hbm, out_hbm, idx_v, rows_v, sem):
    wid = lax.axis_index("s") * _NC + lax.axis_index("c")
    base = wid * _N_PER_W
    pltpu.sync_copy(idx_hbm.at[pl.ds(base, _N_PER_W)], idx_v)

    offs0 = (lax.broadcasted_iota(jnp.int32, (_LANES,), 0) % F) * V
    step = jnp.int32(_LANES % F) * V
    limit = jnp.int32(F) * V

    def body(i, offs):
        sl = pl.ds(i * _LANES, _LANES)
        idx_v[sl] = idx_v[sl] + offs
        nxt = offs + step
        return jnp.where(nxt >= limit, nxt - limit, nxt)

    lax.fori_loop(0, _N_PER_W // _LANES, body, offs0, unroll=4)

    pltpu.async_copy(table_hbm.at[idx_v], rows_v, sem).wait()
    pltpu.sync_copy(rows_v, out_hbm.at[pl.ds(base, _N_PER_W)])


@functools.partial(
    pl.kernel,
    mesh=plsc.VectorSubcoreMesh(core_axis_name="c", subcore_axis_name="s"),
    out_type=jax.ShapeDtypeStruct((B * F, D), jnp.float32),
    scratch_types=[
        pltpu.VMEM((_N_PER_W,), jnp.int32),
        pltpu.VMEM((_N_PER_W, D), jnp.float32),
        pltpu.SemaphoreType.DMA,
    ],
    compiler_params=pltpu.CompilerParams(use_tc_tiling_on_sc=False),
)
def _sc_gather(idx_hbm, table_hbm, out_hbm, idx_v, rows_v, sem):
    _gather_body(idx_hbm, table_hbm, out_hbm, idx_v, rows_v, sem)


def _mlp_body(h_ref, w1_ref, b1_ref, w2_ref, b2_ref, o_ref):
    z = jnp.dot(h_ref[...], w1_ref[...], preferred_element_type=jnp.float32)
    z = jnp.maximum(z + b1_ref[...], 0.0)
    p = jnp.dot(z, w2_ref[...], preferred_element_type=jnp.float32) + b2_ref[...]
    ps = jax.nn.sigmoid(p)
    pctr = ps[:, 0:1]
    o_ref[...] = jnp.concatenate([pctr, pctr * ps[:, 1:2]], axis=1)


def _mlp(h, w1cat, b1cat, w2cat, b2cat, bs=512):
    return pl.pallas_call(
        _mlp_body,
        grid=(B // bs,),
        in_specs=[
            pl.BlockSpec((bs, INPUT_DIM), lambda i: (i, 0)),
            pl.BlockSpec((INPUT_DIM, 2 * H1), lambda i: (0, 0)),
            pl.BlockSpec((1, 2 * H1), lambda i: (0, 0)),
            pl.BlockSpec((2 * H1, 2), lambda i: (0, 0)),
            pl.BlockSpec((1, 2), lambda i: (0, 0)),
        ],
        out_specs=pl.BlockSpec((bs, 2), lambda i: (i, 0)),
        out_shape=jax.ShapeDtypeStruct((B, 2), jnp.float32),
    )(h, w1cat, b1cat, w2cat, b2cat)


def kernel(inputs, tables, ctr_W1, ctr_b1, ctr_W2, ctr_b2,
           cvr_W1, cvr_b1, cvr_W2, cvr_b2):
    idx_flat = inputs.astype(jnp.int32).reshape(-1)
    table_flat = tables.reshape(F * V, D)
    rows = _sc_gather(idx_flat, table_flat)
    h = rows.reshape(B, F * D)

    w1cat = jnp.concatenate([ctr_W1, cvr_W1], axis=1)
    b1cat = jnp.concatenate([ctr_b1, cvr_b1]).reshape(1, -1)
    zeros = jnp.zeros((H1, 1), jnp.float32)
    w2cat = jnp.concatenate(
        [jnp.concatenate([ctr_W2, zeros], axis=0),
         jnp.concatenate([zeros, cvr_W2], axis=0)], axis=1)
    b2cat = jnp.concatenate([ctr_b2, cvr_b2]).reshape(1, 2)

    out2 = _mlp(h, w1cat, b1cat, w2cat, b2cat)
    return out2[:, 0:1], out2[:, 1:2]


# speedup vs baseline: 2.2060x; 1.0007x over previous
import functools

import jax
import jax.numpy as jnp
from jax import lax
from jax.experimental import pallas as pl
from jax.experimental.pallas import tpu as pltpu
from jax.experimental.pallas import tpu_sc as plsc

B, F, V, D = 4096, 26, 100000, 32
INPUT_DIM = F * D
H1 = 128

_SC_INFO = plsc.get_sparse_core_info()
_NC = _SC_INFO.num_cores
_NS = _SC_INFO.num_subcores
_NW = _NC * _NS
_N_PER_W = (B * F) // _NW
_LANES = 16


def _gather_body(idx_hbm, table_hbm, out_hbm, idx_v, rows_v, sem):
    wid = lax.axis_index("s") * _NC + lax.axis_index("c")
    base = wid * _N_PER_W
    pltpu.sync_copy(idx_hbm.at[pl.ds(base, _N_PER_W)], idx_v)

    offs0 = (lax.broadcasted_iota(jnp.int32, (_LANES,), 0) % F) * V
    step = jnp.int32(_LANES % F) * V
    limit = jnp.int32(F) * V

    def body(i, offs):
        sl = pl.ds(i * _LANES, _LANES)
        idx_v[sl] = idx_v[sl] + offs
        nxt = offs + step
        return jnp.where(nxt >= limit, nxt - limit, nxt)

    lax.fori_loop(0, _N_PER_W // _LANES, body, offs0, unroll=4)

    pltpu.async_copy(table_hbm.at[idx_v], rows_v, sem).wait()
    pltpu.sync_copy(rows_v, out_hbm.at[pl.ds(base, _N_PER_W)])


@functools.partial(
    pl.kernel,
    mesh=plsc.VectorSubcoreMesh(core_axis_name="c", subcore_axis_name="s"),
    out_type=jax.ShapeDtypeStruct((B * F, D), jnp.float32),
    scratch_types=[
        pltpu.VMEM((_N_PER_W,), jnp.int32),
        pltpu.VMEM((_N_PER_W, D), jnp.float32),
        pltpu.SemaphoreType.DMA,
    ],
    compiler_params=pltpu.CompilerParams(use_tc_tiling_on_sc=False),
)
def _sc_gather(idx_hbm, table_hbm, out_hbm, idx_v, rows_v, sem):
    _gather_body(idx_hbm, table_hbm, out_hbm, idx_v, rows_v, sem)


def _mlp_body(h_ref, w1_ref, b1_ref, w2_ref, b2_ref, o_ref):
    z = jnp.dot(h_ref[...], w1_ref[...], preferred_element_type=jnp.float32)
    z = jnp.maximum(z + b1_ref[...], 0.0)
    p = jnp.dot(z, w2_ref[...], preferred_element_type=jnp.float32) + b2_ref[...]
    ps = jax.nn.sigmoid(p)
    pctr = ps[:, 0:1]
    o_ref[...] = jnp.concatenate([pctr, pctr * ps[:, 1:2]], axis=1)


def _mlp(h, w1cat, b1cat, w2cat, b2cat, bs=512):
    return pl.pallas_call(
        _mlp_body,
        grid=(B // bs,),
        in_specs=[
            pl.BlockSpec((bs, INPUT_DIM), lambda i: (i, 0)),
            pl.BlockSpec((INPUT_DIM, 2 * H1), lambda i: (0, 0)),
            pl.BlockSpec((1, 2 * H1), lambda i: (0, 0)),
            pl.BlockSpec((2 * H1, 2), lambda i: (0, 0)),
            pl.BlockSpec((1, 2), lambda i: (0, 0)),
        ],
        out_specs=pl.BlockSpec((bs, 2), lambda i: (i, 0)),
        out_shape=jax.ShapeDtypeStruct((B, 2), jnp.float32),
    )(h, w1cat, b1cat, w2cat, b2cat)


def kernel(inputs, tables, ctr_W1, ctr_b1, ctr_W2, ctr_b2,
           cvr_W1, cvr_b1, cvr_W2, cvr_b2):
    idx_flat = inputs.astype(jnp.int32).reshape(-1)
    table_1d = lax.optimization_barrier(tables.reshape(-1))
    table_flat = table_1d.reshape(F * V, D)
    rows = _sc_gather(idx_flat, table_flat)
    h = rows.reshape(B, F * D)

    w1cat = jnp.concatenate([ctr_W1, cvr_W1], axis=1)
    b1cat = jnp.concatenate([ctr_b1, cvr_b1]).reshape(1, -1)
    zeros = jnp.zeros((H1, 1), jnp.float32)
    w2cat = jnp.concatenate(
        [jnp.concatenate([ctr_W2, zeros], axis=0),
         jnp.concatenate([zeros, cvr_W2], axis=0)], axis=1)
    b2cat = jnp.concatenate([ctr_b2, cvr_b2]).reshape(1, 2)

    out2 = _mlp(h, w1cat, b1cat, w2cat, b2cat)
    return out2[:, 0:1], out2[:, 1:2]
